# baseline (speedup 1.0000x reference)
import jax
import jax.numpy as jnp
from jax import lax
from jax.experimental import pallas as pl
from jax.experimental.pallas import tpu as pltpu
from jax.experimental.pallas import tpu_sc as plsc

N = 32768
DIN = 64
H = 16
E = 16
S = 2
K = 2

NC = 2
NS = 16
NW = NC * NS
TPW = N // NW
CH = 256
NGROUP = CH // 16
NCHUNK = TPW // CH
NT = N // 128

BN = 4096
BS = BN * E // 128

_INTERPRET = False
_VARIANT = "full"


_ROT1 = (13, 15, 26, 6)
_ROT2 = (17, 29, 16, 24)


def _threefry_bits(k0, k1, x1):
    u32 = jnp.uint32
    ks2 = k0 ^ k1 ^ u32(0x1BD11BDA)
    x0 = jnp.broadcast_to(k0, x1.shape)
    x1 = x1 + k1
    sched = ((_ROT1, k1, ks2, 1), (_ROT2, ks2, k0, 2), (_ROT1, k0, k1, 3),
             (_ROT2, k1, ks2, 4), (_ROT1, ks2, k0, 5))
    for rots, a0, a1, c in sched:
        for r in rots:
            x0 = x0 + x1
            x1 = (x1 << u32(r)) | (x1 >> u32(32 - r))
            x1 = x1 ^ x0
        x0 = x0 + a0
        x1 = x1 + a1 + u32(c)
    return x0 ^ x1


def _tc_body(keys_ref, xt_ref, w1t_ref, b1c_ref, w2t_ref, b2c_ref, base_ref,
             libt_ref, s0_ref, s1_ref, s2_ref, s3_ref):
    ht = jnp.tanh(
        jnp.dot(w1t_ref[...], xt_ref[...], preferred_element_type=jnp.float32)
        + b1c_ref[...])
    libt_ref[...] = (
        jnp.dot(w2t_ref[...], ht, preferred_element_type=jnp.float32)
        + b2c_ref[...])
    i = pl.program_id(0)
    flat = (lax.broadcasted_iota(jnp.uint32, (BS, 128), 0) * jnp.uint32(128)
            + lax.broadcasted_iota(jnp.uint32, (BS, 128), 1)
            + jnp.uint32(i * (BN * E)))
    fone = jnp.uint32(0x3F800000)
    lo = jnp.float32(1e-10)
    for dj, s_ref in enumerate((s0_ref, s1_ref, s2_ref, s3_ref)):
        bits = _threefry_bits(keys_ref[dj, 0], keys_ref[dj, 1], flat)
        fb = (bits >> jnp.uint32(9)) | fone
        f = lax.bitcast_convert_type(fb, jnp.float32) - jnp.float32(1.0)
        u = jnp.maximum(lo, f * (jnp.float32(1.0) - lo) + lo)
        g = -jnp.log(-jnp.log(u))
        s_ref[...] = g + base_ref[dj][None, :]


def _sc_body(s0_hbm, s1_hbm, s2_hbm, s3_hbm, libt_hbm, coef_hbm,
             gates_hbm, dx_hbm,
             s0a, s1a, s2a, s3a, liba, s0b, s1b, s2b, s3b, libb,
             coef_v,
             g0a, g1a, g2a, g3a, dxa, g0b, g1b, g2b, g3b, dxb,
             sem_ina, sem_inb, sem_outa, sem_outb):
    wid = lax.axis_index("s") * NC + lax.axis_index("c")
    nt0 = wid * NCHUNK
    s_hs = (s0_hbm, s1_hbm, s2_hbm, s3_hbm)
    bufs = (
        ((s0a, s1a, s2a, s3a), liba, (g0a, g1a, g2a, g3a), dxa,
         sem_ina, sem_outa),
        ((s0b, s1b, s2b, s3b), libb, (g0b, g1b, g2b, g3b), dxb,
         sem_inb, sem_outb),
    )

    pltpu.sync_copy(coef_hbm, coef_v)

    lane = lax.iota(jnp.int32, 16)

    def in_descs(nt, b):
        s_vs, lib_v, _, _, sem_in, _ = bufs[b]
        c0 = nt * CH
        ds = [pltpu.make_async_copy(s_hs[dj].at[pl.ds(c0 * E, CH * E)],
                                    s_vs[dj], sem_in) for dj in range(4)]
        ds.append(pltpu.make_async_copy(libt_hbm.at[:, pl.ds(c0, CH)],
                                        lib_v, sem_in))
        return ds

    def out_descs(nt, b):
        _, _, g_vs, dx_v, _, sem_out = bufs[b]
        ds = []
        for dj in range(4):
            for et in range(2):
                ds.append(pltpu.make_async_copy(
                    g_vs[dj].at[pl.ds(et * (CH * 8), CH * 8)],
                    gates_hbm.at[pl.ds(dj * (2 * NT * 1024)
                                       + et * (NT * 1024)
                                       + nt * (CH * 8), CH * 8)],
                    sem_out))
        ds.append(pltpu.make_async_copy(
            dx_v, dx_hbm.at[pl.ds(nt * (S * CH), S * CH)], sem_out))
        return ds

    def compute_chunk(b):
        s_vs, lib_v, g_vs, dx_v, _, _ = bufs[b]

        def group_body(gi, _):
            tpos = gi * 16 + lane
            flat0 = tpos * E
            goff = (gi // 8) * 1024 + (gi % 8) * 16
            sidx = [flat0 + e for e in range(E)]
            for d in range(S):
                idx_prev = None
                dx_d = jnp.zeros((16,), jnp.float32)
                for j in range(K):
                    dj = d * K + j
                    sv = []
                    for e in range(E):
                        col = plsc.load_gather(s_vs[dj], [sidx[e]])
                        if j == 1:
                            col = jnp.where(idx_prev == e,
                                            jnp.float32(-1e9), col)
                        sv.append(col)
                    vals = list(sv)
                    idxs = None
                    while len(vals) > 1:
                        nv, ni = [], []
                        for a in range(0, len(vals), 2):
                            gt = vals[a + 1] > vals[a]
                            nv.append(jnp.where(gt, vals[a + 1], vals[a]))
                            if idxs is None:
                                ni.append(jnp.where(gt, jnp.int32(a + 1),
                                                    jnp.int32(a)))
                            else:
                                ni.append(jnp.where(gt, idxs[a + 1],
                                                    idxs[a]))
                        vals = nv
                        idxs = ni if idxs is None else ni
                    idx = idxs[0]
                    for e in range(E):
                        oh = jnp.where(idx == e, jnp.float32(1.0),
                                       jnp.float32(0.0))
                        off = (e // 8) * (CH * 8) + (e % 8) * 128
                        g_vs[dj][pl.ds(off + goff, 16)] = oh
                    csel = plsc.load_gather(coef_v, [idx + dj * E])
                    lsel = plsc.load_gather(lib_v, [idx, tpos])
                    dx_d = dx_d + csel * lsel
                    idx_prev = idx
                dx_v[pl.ds((gi // 8) * 256 + d * 128 + (gi % 8) * 16, 16)] \
                    = dx_d
            return 0

        lax.fori_loop(0, NGROUP, group_body, 0)

    for dsc in in_descs(nt0, 0):
        dsc.start()

    def pair_body(p, _):
        ntA = nt0 + 2 * p
        ntB = ntA + 1
        for dsc in in_descs(ntB, 1):
            dsc.start()
        for dsc in in_descs(ntA, 0):
            dsc.wait()

        @pl.when(p > 0)
        def _():
            for dsc in out_descs(ntA, 0):
                dsc.wait()
        compute_chunk(0)
        for dsc in out_descs(ntA, 0):
            dsc.start()

        @pl.when(p + 1 < NCHUNK // 2)
        def _():
            for dsc in in_descs(ntA + 2, 0):
                dsc.start()
        for dsc in in_descs(ntB, 1):
            dsc.wait()

        @pl.when(p > 0)
        def _():
            for dsc in out_descs(ntB, 1):
                dsc.wait()
        compute_chunk(1)
        for dsc in out_descs(ntB, 1):
            dsc.start()
        return 0

    lax.fori_loop(0, NCHUNK // 2, pair_body, 0)
    for dsc in out_descs(nt0 + NCHUNK - 2, 0):
        dsc.wait()
    for dsc in out_descs(nt0 + NCHUNK - 1, 1):
        dsc.wait()


def _route_sc(s_flats, libt, coef_flat):
    mesh = plsc.VectorSubcoreMesh(core_axis_name="c", subcore_axis_name="s",
                                  num_cores=NC)
    f = pl.kernel(
        _sc_body,
        out_type=(
            jax.ShapeDtypeStruct((4 * 2 * NT * 1024,), jnp.float32),
            jax.ShapeDtypeStruct((N * S,), jnp.float32),
        ),
        mesh=mesh,
        scratch_types=(
            ([pltpu.VMEM((CH * E,), jnp.float32) for _ in range(4)]
             + [pltpu.VMEM((E, CH), jnp.float32)]) * 2
            + [pltpu.VMEM((S * K * E,), jnp.float32)]
            + ([pltpu.VMEM((2 * CH * 8,), jnp.float32) for _ in range(4)]
               + [pltpu.VMEM((S * CH,), jnp.float32)]) * 2
            + [pltpu.SemaphoreType.DMA] * 4
        ),
        compiler_params=pltpu.CompilerParams(needs_layout_passes=False),
        interpret=_INTERPRET,
    )
    return f(*s_flats, libt, coef_flat)


def kernel(X, W1, b1, W2, b2, routers, coefficients, complexity_prior,
           temperature=1.0, hard=True):
    xt = X.T
    w1t = jnp.transpose(W1, (0, 2, 1)).reshape(E * H, DIN)
    b1c = b1.reshape(E * H, 1)
    w2t = (jnp.eye(E, dtype=W2.dtype)[:, :, None]
           * W2[:, :, 0][None, :, :]).reshape(E, E * H)
    b2c = b2.reshape(E, 1)
    base = routers + complexity_prior[None, :]
    base4 = jnp.concatenate([base[d][None] for d in (0, 0, 1, 1)], 0)
    base_tiled = jnp.tile(base4, (1, 128 // E))

    keys4 = jnp.asarray([[0x4B665424, 0x9617674F],
                         [0xAB7D1D1B, 0x652FBEF2],
                         [0x7DFADB80, 0x23F5531C],
                         [0xD1552267, 0x0859A9E2]], dtype=jnp.uint32)

    grid = (N // BN,)
    libt, s0, s1, s2, s3 = pl.pallas_call(
        _tc_body,
        grid=grid,
        in_specs=[
            pl.BlockSpec(memory_space=pltpu.SMEM),
            pl.BlockSpec((DIN, BN), lambda i: (0, i)),
            pl.BlockSpec((E * H, DIN), lambda i: (0, 0)),
            pl.BlockSpec((E * H, 1), lambda i: (0, 0)),
            pl.BlockSpec((E, E * H), lambda i: (0, 0)),
            pl.BlockSpec((E, 1), lambda i: (0, 0)),
            pl.BlockSpec((4, 128), lambda i: (0, 0)),
        ],
        out_specs=[
            pl.BlockSpec((E, BN), lambda i: (0, i)),
        ] + [pl.BlockSpec((BS, 128), lambda i: (i, 0))] * 4,
        out_shape=[
            jax.ShapeDtypeStruct((E, N), jnp.float32),
        ] + [jax.ShapeDtypeStruct((N * E // 128, 128), jnp.float32)] * 4,
        interpret=_INTERPRET,
    )(keys4, xt, w1t, b1c, w2t, b2c, base_tiled)

    if _VARIANT == "tc_only":
        dxdt = libt[:S, :].T * 0.0
        gates4 = jnp.zeros((S, K, N, E), jnp.float32) + s0[0, 0]
        return dxdt, gates4

    coef_flat = coefficients.reshape(S * K * E)
    s_flats = [s.reshape(N * E) for s in (s0, s1, s2, s3)]
    gates_flat, dx_flat = _route_sc(s_flats, libt, coef_flat)

    gates = (gates_flat.reshape(4, 2, NT, 8, 128)
             .transpose(0, 1, 3, 2, 4)
             .reshape(S, K, E, N)
             .transpose(0, 1, 3, 2))
    dxdt = (dx_flat.reshape(NT, S, 128)
            .transpose(1, 0, 2)
            .reshape(S, N)
            .transpose(1, 0))
    return dxdt, gates


# speedup vs baseline: 1.0769x; 1.0769x over previous
import jax
import jax.numpy as jnp
from jax import lax
from jax.experimental import pallas as pl
from jax.experimental.pallas import tpu as pltpu
from jax.experimental.pallas import tpu_sc as plsc

N = 32768
DIN = 64
H = 16
E = 16
S = 2
K = 2

NC = 2
NS = 16
NW = NC * NS
TPW = N // NW
CH = 256
NGROUP = CH // 16
NCHUNK = TPW // CH
NT = N // 128

BN = 2048
BS = BN * E // 128

_INTERPRET = False
_VARIANT = "full"


_ROT1 = (13, 15, 26, 6)
_ROT2 = (17, 29, 16, 24)


def _threefry_bits(k0, k1, x1):
    u32 = jnp.uint32
    ks2 = k0 ^ k1 ^ u32(0x1BD11BDA)
    x0 = jnp.broadcast_to(k0, x1.shape)
    x1 = x1 + k1
    sched = ((_ROT1, k1, ks2, 1), (_ROT2, ks2, k0, 2), (_ROT1, k0, k1, 3),
             (_ROT2, k1, ks2, 4), (_ROT1, ks2, k0, 5))
    for rots, a0, a1, c in sched:
        for r in rots:
            x0 = x0 + x1
            x1 = (x1 << u32(r)) | (x1 >> u32(32 - r))
            x1 = x1 ^ x0
        x0 = x0 + a0
        x1 = x1 + a1 + u32(c)
    return x0 ^ x1


def _tc_body(keys_ref, xt_ref, w1t_ref, b1c_ref, w2t_ref, b2c_ref, base_ref,
             libt_ref, s0_ref, s1_ref, s2_ref, s3_ref):
    ht = jnp.tanh(
        jnp.dot(w1t_ref[...], xt_ref[...], preferred_element_type=jnp.float32)
        + b1c_ref[...])
    libt_ref[...] = (
        jnp.dot(w2t_ref[...], ht, preferred_element_type=jnp.float32)
        + b2c_ref[...])
    i = pl.program_id(0)
    flat = (lax.broadcasted_iota(jnp.uint32, (BS, 128), 0) * jnp.uint32(128)
            + lax.broadcasted_iota(jnp.uint32, (BS, 128), 1)
            + jnp.uint32(i * (BN * E)))
    fone = jnp.uint32(0x3F800000)
    lo = jnp.float32(1e-10)
    for dj, s_ref in enumerate((s0_ref, s1_ref, s2_ref, s3_ref)):
        bits = _threefry_bits(keys_ref[dj, 0], keys_ref[dj, 1], flat)
        fb = (bits >> jnp.uint32(9)) | fone
        f = lax.bitcast_convert_type(fb, jnp.float32) - jnp.float32(1.0)
        u = jnp.maximum(lo, f * (jnp.float32(1.0) - lo) + lo)
        g = -jnp.log(-jnp.log(u))
        s_ref[...] = g + base_ref[dj][None, :]


def _sc_body(s0_hbm, s1_hbm, s2_hbm, s3_hbm, libt_hbm, coef_hbm,
             gates_hbm, dx_hbm,
             s0a, s1a, s2a, s3a, liba, s0b, s1b, s2b, s3b, libb,
             coef_v,
             g0a, g1a, g2a, g3a, dxa, g0b, g1b, g2b, g3b, dxb,
             sem_ina, sem_inb, sem_outa, sem_outb):
    wid = lax.axis_index("s") * NC + lax.axis_index("c")
    nt0 = wid * NCHUNK
    s_hs = (s0_hbm, s1_hbm, s2_hbm, s3_hbm)
    bufs = (
        ((s0a, s1a, s2a, s3a), liba, (g0a, g1a, g2a, g3a), dxa,
         sem_ina, sem_outa),
        ((s0b, s1b, s2b, s3b), libb, (g0b, g1b, g2b, g3b), dxb,
         sem_inb, sem_outb),
    )

    pltpu.sync_copy(coef_hbm, coef_v)

    lane = lax.iota(jnp.int32, 16)

    def in_descs(nt, b):
        s_vs, lib_v, _, _, sem_in, _ = bufs[b]
        c0 = nt * CH
        ds = [pltpu.make_async_copy(s_hs[dj].at[pl.ds(c0 * E, CH * E)],
                                    s_vs[dj], sem_in) for dj in range(4)]
        ds.append(pltpu.make_async_copy(libt_hbm.at[:, pl.ds(c0, CH)],
                                        lib_v, sem_in))
        return ds

    def out_descs(nt, b):
        _, _, g_vs, dx_v, _, sem_out = bufs[b]
        ds = []
        for dj in range(4):
            for et in range(2):
                ds.append(pltpu.make_async_copy(
                    g_vs[dj].at[pl.ds(et * (CH * 8), CH * 8)],
                    gates_hbm.at[pl.ds(dj * (2 * NT * 1024)
                                       + et * (NT * 1024)
                                       + nt * (CH * 8), CH * 8)],
                    sem_out))
        ds.append(pltpu.make_async_copy(
            dx_v, dx_hbm.at[pl.ds(nt * (S * CH), S * CH)], sem_out))
        return ds

    def compute_chunk(b):
        s_vs, lib_v, g_vs, dx_v, _, _ = bufs[b]

        def group_body(gi, _):
            tpos = gi * 16 + lane
            flat0 = tpos * E
            goff = (gi // 8) * 1024 + (gi % 8) * 16
            sidx = [flat0 + e for e in range(E)]
            for d in range(S):
                idx_prev = None
                dx_d = jnp.zeros((16,), jnp.float32)
                for j in range(K):
                    dj = d * K + j
                    sv = []
                    for e in range(E):
                        col = plsc.load_gather(s_vs[dj], [sidx[e]])
                        if j == 1:
                            col = jnp.where(idx_prev == e,
                                            jnp.float32(-1e9), col)
                        sv.append(col)
                    vals = list(sv)
                    idxs = None
                    while len(vals) > 1:
                        nv, ni = [], []
                        for a in range(0, len(vals), 2):
                            gt = vals[a + 1] > vals[a]
                            nv.append(jnp.where(gt, vals[a + 1], vals[a]))
                            if idxs is None:
                                ni.append(jnp.where(gt, jnp.int32(a + 1),
                                                    jnp.int32(a)))
                            else:
                                ni.append(jnp.where(gt, idxs[a + 1],
                                                    idxs[a]))
                        vals = nv
                        idxs = ni if idxs is None else ni
                    idx = idxs[0]
                    for e in range(E):
                        oh = jnp.where(idx == e, jnp.float32(1.0),
                                       jnp.float32(0.0))
                        off = (e // 8) * (CH * 8) + (e % 8) * 128
                        g_vs[dj][pl.ds(off + goff, 16)] = oh
                    csel = plsc.load_gather(coef_v, [idx + dj * E])
                    lsel = plsc.load_gather(lib_v, [idx, tpos])
                    dx_d = dx_d + csel * lsel
                    idx_prev = idx
                dx_v[pl.ds((gi // 8) * 256 + d * 128 + (gi % 8) * 16, 16)] \
                    = dx_d
            return 0

        lax.fori_loop(0, NGROUP, group_body, 0)

    for dsc in in_descs(nt0, 0):
        dsc.start()

    def pair_body(p, _):
        ntA = nt0 + 2 * p
        ntB = ntA + 1
        for dsc in in_descs(ntB, 1):
            dsc.start()
        for dsc in in_descs(ntA, 0):
            dsc.wait()

        @pl.when(p > 0)
        def _():
            for dsc in out_descs(ntA, 0):
                dsc.wait()
        compute_chunk(0)
        for dsc in out_descs(ntA, 0):
            dsc.start()

        @pl.when(p + 1 < NCHUNK // 2)
        def _():
            for dsc in in_descs(ntA + 2, 0):
                dsc.start()
        for dsc in in_descs(ntB, 1):
            dsc.wait()

        @pl.when(p > 0)
        def _():
            for dsc in out_descs(ntB, 1):
                dsc.wait()
        compute_chunk(1)
        for dsc in out_descs(ntB, 1):
            dsc.start()
        return 0

    lax.fori_loop(0, NCHUNK // 2, pair_body, 0)
    for dsc in out_descs(nt0 + NCHUNK - 2, 0):
        dsc.wait()
    for dsc in out_descs(nt0 + NCHUNK - 1, 1):
        dsc.wait()


def _route_sc(s_flats, libt, coef_flat):
    mesh = plsc.VectorSubcoreMesh(core_axis_name="c", subcore_axis_name="s",
                                  num_cores=NC)
    f = pl.kernel(
        _sc_body,
        out_type=(
            jax.ShapeDtypeStruct((4 * 2 * NT * 1024,), jnp.float32),
            jax.ShapeDtypeStruct((N * S,), jnp.float32),
        ),
        mesh=mesh,
        scratch_types=(
            ([pltpu.VMEM((CH * E,), jnp.float32) for _ in range(4)]
             + [pltpu.VMEM((E, CH), jnp.float32)]) * 2
            + [pltpu.VMEM((S * K * E,), jnp.float32)]
            + ([pltpu.VMEM((2 * CH * 8,), jnp.float32) for _ in range(4)]
               + [pltpu.VMEM((S * CH,), jnp.float32)]) * 2
            + [pltpu.SemaphoreType.DMA] * 4
        ),
        compiler_params=pltpu.CompilerParams(needs_layout_passes=False),
        interpret=_INTERPRET,
    )
    return f(*s_flats, libt, coef_flat)


def kernel(X, W1, b1, W2, b2, routers, coefficients, complexity_prior,
           temperature=1.0, hard=True):
    xt = X.T
    w1t = jnp.transpose(W1, (0, 2, 1)).reshape(E * H, DIN)
    b1c = b1.reshape(E * H, 1)
    w2t = (jnp.eye(E, dtype=W2.dtype)[:, :, None]
           * W2[:, :, 0][None, :, :]).reshape(E, E * H)
    b2c = b2.reshape(E, 1)
    base = routers + complexity_prior[None, :]
    base4 = jnp.concatenate([base[d][None] for d in (0, 0, 1, 1)], 0)
    base_tiled = jnp.tile(base4, (1, 128 // E))

    keys4 = jnp.asarray([[0x4B665424, 0x9617674F],
                         [0xAB7D1D1B, 0x652FBEF2],
                         [0x7DFADB80, 0x23F5531C],
                         [0xD1552267, 0x0859A9E2]], dtype=jnp.uint32)

    grid = (N // BN,)
    libt, s0, s1, s2, s3 = pl.pallas_call(
        _tc_body,
        grid=grid,
        in_specs=[
            pl.BlockSpec(memory_space=pltpu.SMEM),
            pl.BlockSpec((DIN, BN), lambda i: (0, i)),
            pl.BlockSpec((E * H, DIN), lambda i: (0, 0)),
            pl.BlockSpec((E * H, 1), lambda i: (0, 0)),
            pl.BlockSpec((E, E * H), lambda i: (0, 0)),
            pl.BlockSpec((E, 1), lambda i: (0, 0)),
            pl.BlockSpec((4, 128), lambda i: (0, 0)),
        ],
        out_specs=[
            pl.BlockSpec((E, BN), lambda i: (0, i)),
        ] + [pl.BlockSpec((BS, 128), lambda i: (i, 0))] * 4,
        out_shape=[
            jax.ShapeDtypeStruct((E, N), jnp.float32),
        ] + [jax.ShapeDtypeStruct((N * E // 128, 128), jnp.float32)] * 4,
        interpret=_INTERPRET,
    )(keys4, xt, w1t, b1c, w2t, b2c, base_tiled)

    if _VARIANT == "tc_only":
        dxdt = libt[:S, :].T * 0.0
        gates4 = jnp.zeros((S, K, N, E), jnp.float32) + s0[0, 0]
        return dxdt, gates4

    coef_flat = coefficients.reshape(S * K * E)
    s_flats = [s.reshape(N * E) for s in (s0, s1, s2, s3)]
    gates_flat, dx_flat = _route_sc(s_flats, libt, coef_flat)

    gates = (gates_flat.reshape(4, 2, NT, 8, 128)
             .transpose(0, 1, 3, 2, 4)
             .reshape(S, K, E, N)
             .transpose(0, 1, 3, 2))
    dxdt = (dx_flat.reshape(NT, S, 128)
            .transpose(1, 0, 2)
            .reshape(S, N)
            .transpose(1, 0))
    return dxdt, gates
